# baseline (speedup 1.0000x reference)
import functools

import jax
import jax.numpy as jnp
from jax import lax
from jax.experimental import pallas as pl
from jax.experimental.pallas import tpu as pltpu
from jax.experimental.pallas import tpu_sc as plsc

NUM_ITEMS = 1000000
D = 32
B = 16384
NC = 2
NS = 16
NW = NC * NS
B_PER_W = B // NW
CHUNK = 128
N_CHUNKS = B_PER_W // CHUNK
GROUPS = B_PER_W // 16


def _lane_perm(t, p):
    dnums = lax.GatherDimensionNumbers(
        offset_dims=(), collapsed_slice_dims=(0,), start_index_map=(0,))
    return lax.gather(t, p[:, None], dnums, slice_sizes=(1,),
                      mode=lax.GatherScatterMode.PROMISE_IN_BOUNDS)


def _make_sc_kernel():
    mesh = plsc.VectorSubcoreMesh(core_axis_name="c", subcore_axis_name="s")

    @functools.partial(
        pl.kernel,
        mesh=mesh,
        compiler_params=pltpu.CompilerParams(use_tc_tiling_on_sc=False),
        out_type=jax.ShapeDtypeStruct((B,), jnp.float32),
        scratch_types=[
            pltpu.VMEM((N_CHUNKS, CHUNK), jnp.int32),
            pltpu.VMEM((B_PER_W, D), jnp.float32),
            pltpu.VMEM((D,), jnp.float32),
            pltpu.VMEM((B_PER_W,), jnp.float32),
            pltpu.SemaphoreType.DMA,
        ],
    )
    def fcf_kernel(idx_hbm, table_hbm, u_hbm, out_hbm,
                   idx_v, rows_v, u_v, out_v, sem):
        wid = lax.axis_index("s") * NC + lax.axis_index("c")
        base = wid * B_PER_W

        pltpu.sync_copy(idx_hbm.at[wid], idx_v)
        pltpu.sync_copy(u_hbm, u_v)

        copies = []
        for j in range(N_CHUNKS):
            copies.append(pltpu.async_copy(
                table_hbm.at[idx_v.at[j]],
                rows_v.at[pl.ds(j * CHUNK, CHUNK)],
                sem,
            ))
        for c in copies:
            c.wait()

        u_lo = u_v[pl.ds(0, 16)]
        u_hi = u_v[pl.ds(16, 16)]
        lane = lax.iota(jnp.int32, 16)
        perms = [lane ^ jnp.int32(s) for s in (1, 2, 4, 8)]

        def group_body(g, carry):
            acc = jnp.zeros((16,), jnp.float32)
            for i in range(16):
                row = g * 16 + i
                t = (rows_v[row, pl.ds(0, 16)] * u_lo
                     + rows_v[row, pl.ds(16, 16)] * u_hi)
                for p in perms:
                    t = t + _lane_perm(t, p)
                acc = jnp.where(lane == i, t, acc)
            out_v[pl.ds(g * 16, 16)] = 1.0 / (1.0 + jnp.exp(-acc))
            return carry

        lax.fori_loop(0, GROUPS, group_body, jnp.int32(0))

        pltpu.sync_copy(out_v, out_hbm.at[pl.ds(base, B_PER_W)])

    return fcf_kernel


_fcf_sc = _make_sc_kernel()


def kernel(item_indices, item_table, user_embedding):
    idx = item_indices.astype(jnp.int32).reshape(NW, N_CHUNKS, CHUNK)
    u = user_embedding.reshape(D)
    return _fcf_sc(idx, item_table, u)
